# baseline (speedup 1.0000x reference)
import jax
import jax.numpy as jnp
from jax.experimental import pallas as pl
from jax.experimental.pallas import tpu as pltpu

_BLK = 1024


def _body(ind_ref, x_ref, w_ref, b_ref, o_ref):
    x = x_ref[...]
    acc = jnp.dot(x, w_ref[...], preferred_element_type=jnp.float32)
    acc = acc + b_ref[...]
    mask_row = (ind_ref[0] != 0).astype(jnp.float32)
    mask_col = jnp.transpose(mask_row)
    o_ref[...] = acc * mask_col


def kernel(indicators, data, W, b):
    N, D = data.shape
    nb = N // _BLK
    ind3 = indicators.reshape(nb, 1, _BLK)
    b2 = b.reshape(1, D)
    return pl.pallas_call(
        _body,
        grid=(nb,),
        in_specs=[
            pl.BlockSpec((1, 1, _BLK), lambda i: (i, 0, 0)),
            pl.BlockSpec((_BLK, D), lambda i: (i, 0)),
            pl.BlockSpec((D, D), lambda i: (0, 0)),
            pl.BlockSpec((1, D), lambda i: (0, 0)),
        ],
        out_specs=pl.BlockSpec((_BLK, D), lambda i: (i, 0)),
        out_shape=jax.ShapeDtypeStruct((N, D), jnp.float32),
        compiler_params=pltpu.CompilerParams(
            dimension_semantics=("parallel",),
        ),
    )(ind3, data, W, b2)


# speedup vs baseline: 1.1745x; 1.1745x over previous
import jax
import jax.numpy as jnp
from jax.experimental import pallas as pl
from jax.experimental.pallas import tpu as pltpu

_N, _D = 16384, 64
_NC = 16
_CH = _N // _NC


def _body(ind_hbm, x_hbm, w_hbm, b_hbm, o_hbm,
          ind_v, w_v, b_v, xbuf, obuf,
          insem, outsem, csem):
    pltpu.make_async_copy(w_hbm, w_v, csem.at[0]).start()
    pltpu.make_async_copy(b_hbm, b_v, csem.at[1]).start()
    pltpu.make_async_copy(ind_hbm, ind_v, csem.at[2]).start()
    for c in range(_NC):
        pltpu.make_async_copy(
            x_hbm.at[pl.ds(c * _CH, _CH)], xbuf.at[c], insem.at[c]
        ).start()
    pltpu.make_async_copy(w_hbm, w_v, csem.at[0]).wait()
    pltpu.make_async_copy(b_hbm, b_v, csem.at[1]).wait()
    pltpu.make_async_copy(ind_hbm, ind_v, csem.at[2]).wait()
    w = w_v[...]
    bias = b_v[...]
    for c in range(_NC):
        pltpu.make_async_copy(
            x_hbm.at[pl.ds(c * _CH, _CH)], xbuf.at[c], insem.at[c]
        ).wait()
        acc = jnp.dot(xbuf[c], w, preferred_element_type=jnp.float32) + bias
        mrow = jnp.where(ind_v[:, pl.ds(c * _CH, _CH)] != 0, 1.0, 0.0)
        obuf[c] = acc * jnp.transpose(mrow)
        pltpu.make_async_copy(
            obuf.at[c], o_hbm.at[pl.ds(c * _CH, _CH)], outsem.at[c]
        ).start()
    for c in range(_NC):
        pltpu.make_async_copy(
            obuf.at[c], o_hbm.at[pl.ds(c * _CH, _CH)], outsem.at[c]
        ).wait()


def kernel(indicators, data, W, b):
    N, D = data.shape
    return pl.pallas_call(
        _body,
        in_specs=[
            pl.BlockSpec(memory_space=pl.ANY),
            pl.BlockSpec(memory_space=pl.ANY),
            pl.BlockSpec(memory_space=pl.ANY),
            pl.BlockSpec(memory_space=pl.ANY),
        ],
        out_specs=pl.BlockSpec(memory_space=pl.ANY),
        out_shape=jax.ShapeDtypeStruct((N, D), jnp.float32),
        scratch_shapes=[
            pltpu.VMEM((1, _N), jnp.int32),
            pltpu.VMEM((D, D), jnp.float32),
            pltpu.VMEM((1, D), jnp.float32),
            pltpu.VMEM((_NC, _CH, _D), jnp.float32),
            pltpu.VMEM((_NC, _CH, _D), jnp.float32),
            pltpu.SemaphoreType.DMA((_NC,)),
            pltpu.SemaphoreType.DMA((_NC,)),
            pltpu.SemaphoreType.DMA((3,)),
        ],
    )(indicators.reshape(1, N), data, W, b.reshape(1, D))
